# baseline (speedup 1.0000x reference)
import functools

import jax
import jax.numpy as jnp
import numpy as np
from jax import lax
from jax.experimental import pallas as pl
from jax.experimental.pallas import tpu as pltpu
from jax.experimental.pallas import tpu_sc as plsc

_RADIUS = 0.05
_R2 = np.float32(_RADIUS * _RADIUS)
_B, _N, _D = 8, 2048, 3
_WORDS = _N // 32
_NW = 32
_G = 4

_SB = 2
_TB = _B - _SB
_ROW_TILE = 256

_CHUNKS = _NW // _SB
_ROWS_PER_W = _N // _CHUNKS
_SUPER = _ROWS_PER_W // (16 * _G)
_OUT_PER_W = _ROWS_PER_W * _WORDS


def _bf16_rne(v):
    u = lax.bitcast_convert_type(v, jnp.int32)
    lsb = lax.shift_right_logical(u, 16) & 1
    r = (u + (0x7FFF + lsb)) & jnp.int32(-65536)
    return lax.bitcast_convert_type(r, jnp.float32)


def _sc_body(locs_hbm, out_hbm, locs_v, out_v, p_x2, p_y2, p_z2, p_sq, sem):
    wid = lax.axis_index("s") * 2 + lax.axis_index("c")
    batch = wid // _CHUNKS
    chunk = wid % _CHUNKS
    lbase = batch * (_D * _N)

    pltpu.sync_copy(locs_hbm.at[pl.ds(lbase, _D * _N)], locs_v)

    lane = lax.iota(jnp.int32, 16)
    lane3 = lane * _D

    def prep(i, c0):
        sl = pl.ds(i * 16, 16)
        x = plsc.load_gather(locs_v, [lane3 + (i * 16 * _D)])
        y = plsc.load_gather(locs_v, [lane3 + (i * 16 * _D + 1)])
        z = plsc.load_gather(locs_v, [lane3 + (i * 16 * _D + 2)])
        xb = _bf16_rne(x)
        yb = _bf16_rne(y)
        zb = _bf16_rne(z)
        p_x2[sl] = xb + xb
        p_y2[sl] = yb + yb
        p_z2[sl] = zb + zb
        p_sq[sl] = (x * x + y * y) + z * z
        return c0
    lax.fori_loop(0, _N // 16, prep, 0)

    weights = [np.float32(2.0 ** t) for t in range(32)]

    def super_group(sg, c1):
        xi, yi, zi, sqi, rbase = [], [], [], [], []
        for g in range(_G):
            base = chunk * _ROWS_PER_W + sg * (16 * _G) + g * 16
            xi.append(_bf16_rne(plsc.load_gather(locs_v, [lane3 + base * _D])))
            yi.append(_bf16_rne(
                plsc.load_gather(locs_v, [lane3 + (base * _D + 1)])))
            zi.append(_bf16_rne(
                plsc.load_gather(locs_v, [lane3 + (base * _D + 2)])))
            sqi.append(p_sq[pl.ds(base, 16)])
            rbase.append((lane + sg * (16 * _G) + g * 16) * _WORDS)

        def word(w, c2):
            j0 = w * 32
            xv = [p_x2[pl.ds(j0, 16)], p_x2[pl.ds(j0 + 16, 16)]]
            yv = [p_y2[pl.ds(j0, 16)], p_y2[pl.ds(j0 + 16, 16)]]
            zv = [p_z2[pl.ds(j0, 16)], p_z2[pl.ds(j0 + 16, 16)]]
            sv = [p_sq[pl.ds(j0, 16)], p_sq[pl.ds(j0 + 16, 16)]]
            acc = [jnp.zeros((16,), jnp.float32) for _ in range(_G)]
            for t in range(32):
                h, l = divmod(t, 16)
                x2 = xv[h][l]
                y2 = yv[h][l]
                z2 = zv[h][l]
                sqj = sv[h][l]
                for g in range(_G):
                    dot2 = (xi[g] * x2 + yi[g] * y2) + zi[g] * z2
                    d2 = (sqi[g] + sqj) - dot2
                    acc[g] = jnp.where(d2 <= _R2, acc[g] + weights[t], acc[g])
            for g in range(_G):
                plsc.store_scatter(out_v, [rbase[g] + w], acc[g])
            return c2
        lax.fori_loop(0, _WORDS, word, 0)
        return c1
    lax.fori_loop(0, _SUPER, super_group, 0)

    pltpu.sync_copy(out_v, out_hbm.at[pl.ds(wid * _OUT_PER_W, _OUT_PER_W)])


def _sc_run(locs_sc):
    locs_t = locs_sc.reshape(_SB * _D * _N)
    mesh = plsc.VectorSubcoreMesh(core_axis_name="c", subcore_axis_name="s")
    run = pl.kernel(
        _sc_body,
        out_type=jax.ShapeDtypeStruct((_NW * _OUT_PER_W,), jnp.float32),
        mesh=mesh,
        compiler_params=pltpu.CompilerParams(needs_layout_passes=False),
        scratch_types=[
            pltpu.VMEM((_D * _N,), jnp.float32),
            pltpu.VMEM((_OUT_PER_W,), jnp.float32),
        ] + [pltpu.VMEM((_N,), jnp.float32)] * 4 + [
            pltpu.SemaphoreType.DMA,
        ],
    )
    return run(locs_t).reshape(_SB, _N, _WORDS)


def _tc_body(locs_ref, pack_ref, out_ref):
    rt = pl.program_id(1)
    a = locs_ref[0]
    x = a[:, 0]
    y = a[:, 1]
    z = a[:, 2]
    sq = (x * x + y * y) + z * z
    lhs = locs_ref[0, pl.ds(rt * _ROW_TILE, _ROW_TILE), :]
    lx = lhs[:, 0]
    ly = lhs[:, 1]
    lz = lhs[:, 2]
    sqi = (lx * lx + ly * ly) + lz * lz
    dots = lax.dot_general(
        lhs.astype(jnp.bfloat16), a.astype(jnp.bfloat16),
        (((1,), (1,)), ((), ())),
        preferred_element_type=jnp.float32)
    d2 = (sqi[:, None] + sq[None, :]) - 2.0 * dots
    mask = (d2 <= _R2).astype(jnp.float32)
    out_ref[0] = lax.dot_general(
        mask, pack_ref[...], (((1,), (0,)), ((), ())),
        preferred_element_type=jnp.float32)


def _tc_run(locs_tc, pack):
    grid = (_TB, _N // _ROW_TILE)
    return pl.pallas_call(
        _tc_body,
        grid=grid,
        in_specs=[
            pl.BlockSpec((1, _N, _D), lambda b, rt: (b, 0, 0)),
            pl.BlockSpec((_N, _WORDS), lambda b, rt: (0, 0)),
        ],
        out_specs=pl.BlockSpec((1, _ROW_TILE, _WORDS),
                               lambda b, rt: (b, rt, 0)),
        out_shape=jax.ShapeDtypeStruct((_TB, _N, _WORDS), jnp.float32),
    )(locs_tc, pack)


def _pack_matrix():
    j = np.arange(_N)
    p = np.zeros((_N, _WORDS), np.float32)
    p[j, j // 32] = (2.0 ** (j % 32)).astype(np.float32)
    return p


_PACK = _pack_matrix()


def kernel(locs):
    out_sc = _sc_run(locs[_TB:])
    out_tc = _tc_run(locs[:_TB], jnp.asarray(_PACK))
    return jnp.concatenate([out_tc, out_sc], axis=0)


# speedup vs baseline: 1.2375x; 1.2375x over previous
import functools

import jax
import jax.numpy as jnp
import numpy as np
from jax import lax
from jax.experimental import pallas as pl
from jax.experimental.pallas import tpu as pltpu
from jax.experimental.pallas import tpu_sc as plsc

_RADIUS = 0.05
_R2 = np.float32(_RADIUS * _RADIUS)
_B, _N, _D = 8, 2048, 3
_WORDS = _N // 32
_NW = 32
_G = 4

_SB = 2
_TB = _B - _SB
_ROW_TILE = 512

_CHUNKS = _NW // _SB
_ROWS_PER_W = _N // _CHUNKS
_SUPER = _ROWS_PER_W // (16 * _G)
_OUT_PER_W = _ROWS_PER_W * _WORDS


def _bf16_rne(v):
    u = lax.bitcast_convert_type(v, jnp.int32)
    lsb = lax.shift_right_logical(u, 16) & 1
    r = (u + (0x7FFF + lsb)) & jnp.int32(-65536)
    return lax.bitcast_convert_type(r, jnp.float32)


def _sc_body(locs_hbm, out_hbm, locs_v, out_v, p_x2, p_y2, p_z2, p_sq, sem):
    wid = lax.axis_index("s") * 2 + lax.axis_index("c")
    batch = wid // _CHUNKS
    chunk = wid % _CHUNKS
    lbase = (_TB + batch) * (_D * _N)

    pltpu.sync_copy(locs_hbm.at[pl.ds(lbase, _D * _N)], locs_v)

    lane = lax.iota(jnp.int32, 16)
    lane3 = lane * _D

    def prep(i, c0):
        sl = pl.ds(i * 16, 16)
        x = plsc.load_gather(locs_v, [lane3 + (i * 16 * _D)])
        y = plsc.load_gather(locs_v, [lane3 + (i * 16 * _D + 1)])
        z = plsc.load_gather(locs_v, [lane3 + (i * 16 * _D + 2)])
        xb = _bf16_rne(x)
        yb = _bf16_rne(y)
        zb = _bf16_rne(z)
        p_x2[sl] = xb + xb
        p_y2[sl] = yb + yb
        p_z2[sl] = zb + zb
        p_sq[sl] = (x * x + y * y) + z * z
        return c0
    lax.fori_loop(0, _N // 16, prep, 0)

    weights = [np.float32(2.0 ** t) for t in range(32)]

    def super_group(sg, c1):
        xi, yi, zi, sqi, rbase = [], [], [], [], []
        for g in range(_G):
            base = chunk * _ROWS_PER_W + sg * (16 * _G) + g * 16
            xi.append(_bf16_rne(plsc.load_gather(locs_v, [lane3 + base * _D])))
            yi.append(_bf16_rne(
                plsc.load_gather(locs_v, [lane3 + (base * _D + 1)])))
            zi.append(_bf16_rne(
                plsc.load_gather(locs_v, [lane3 + (base * _D + 2)])))
            sqi.append(p_sq[pl.ds(base, 16)])
            rbase.append((lane + sg * (16 * _G) + g * 16) * _WORDS)

        def word(w, c2):
            j0 = w * 32
            xv = [p_x2[pl.ds(j0, 16)], p_x2[pl.ds(j0 + 16, 16)]]
            yv = [p_y2[pl.ds(j0, 16)], p_y2[pl.ds(j0 + 16, 16)]]
            zv = [p_z2[pl.ds(j0, 16)], p_z2[pl.ds(j0 + 16, 16)]]
            sv = [p_sq[pl.ds(j0, 16)], p_sq[pl.ds(j0 + 16, 16)]]
            acc = [jnp.zeros((16,), jnp.float32) for _ in range(_G)]
            for t in range(32):
                h, l = divmod(t, 16)
                x2 = xv[h][l]
                y2 = yv[h][l]
                z2 = zv[h][l]
                sqj = sv[h][l]
                for g in range(_G):
                    dot2 = (xi[g] * x2 + yi[g] * y2) + zi[g] * z2
                    d2 = (sqi[g] + sqj) - dot2
                    acc[g] = jnp.where(d2 <= _R2, acc[g] + weights[t], acc[g])
            for g in range(_G):
                plsc.store_scatter(out_v, [rbase[g] + w], acc[g])
            return c2
        lax.fori_loop(0, _WORDS, word, 0)
        return c1
    lax.fori_loop(0, _SUPER, super_group, 0)

    pltpu.sync_copy(out_v, out_hbm.at[pl.ds(wid * _OUT_PER_W, _OUT_PER_W)])


def _sc_run(locs):
    locs_t = locs.reshape(_B * _D * _N)
    mesh = plsc.VectorSubcoreMesh(core_axis_name="c", subcore_axis_name="s")
    run = pl.kernel(
        _sc_body,
        out_type=jax.ShapeDtypeStruct((_NW * _OUT_PER_W,), jnp.float32),
        mesh=mesh,
        compiler_params=pltpu.CompilerParams(needs_layout_passes=False),
        scratch_types=[
            pltpu.VMEM((_D * _N,), jnp.float32),
            pltpu.VMEM((_OUT_PER_W,), jnp.float32),
        ] + [pltpu.VMEM((_N,), jnp.float32)] * 4 + [
            pltpu.SemaphoreType.DMA,
        ],
    )
    return run(locs_t).reshape(_SB, _N, _WORDS)


def _tc_body(locs_ref, pack_ref, out_ref):
    rt = pl.program_id(1)
    a = locs_ref[0]
    x = a[:, 0]
    y = a[:, 1]
    z = a[:, 2]
    sq = (x * x + y * y) + z * z
    lhs = locs_ref[0, pl.ds(rt * _ROW_TILE, _ROW_TILE), :]
    lx = lhs[:, 0]
    ly = lhs[:, 1]
    lz = lhs[:, 2]
    sqi = (lx * lx + ly * ly) + lz * lz
    dots = lax.dot_general(
        lhs.astype(jnp.bfloat16), a.astype(jnp.bfloat16),
        (((1,), (1,)), ((), ())),
        preferred_element_type=jnp.float32)
    d2 = (sqi[:, None] + sq[None, :]) - 2.0 * dots
    mask = (d2 <= _R2).astype(jnp.float32)
    out_ref[0] = lax.dot_general(
        mask, pack_ref[...], (((1,), (0,)), ((), ())),
        preferred_element_type=jnp.float32)


def _tc_run(locs, pack):
    grid = (_TB, _N // _ROW_TILE)
    return pl.pallas_call(
        _tc_body,
        grid=grid,
        in_specs=[
            pl.BlockSpec((1, _N, _D), lambda b, rt: (b, 0, 0)),
            pl.BlockSpec((_N, _WORDS), lambda b, rt: (0, 0)),
        ],
        out_specs=pl.BlockSpec((1, _ROW_TILE, _WORDS),
                               lambda b, rt: (b, rt, 0)),
        out_shape=jax.ShapeDtypeStruct((_TB, _N, _WORDS), jnp.float32),
    )(locs, pack)


def _pack_matrix():
    j = np.arange(_N)
    p = np.zeros((_N, _WORDS), np.float32)
    p[j, j // 32] = (2.0 ** (j % 32)).astype(np.float32)
    return p


_PACK = _pack_matrix()


def kernel(locs):
    out_sc = _sc_run(locs)
    out_tc = _tc_run(locs, jnp.asarray(_PACK))
    return jnp.concatenate([out_tc, out_sc], axis=0)


# speedup vs baseline: 1.4113x; 1.1404x over previous
import functools

import jax
import jax.numpy as jnp
import numpy as np
from jax import lax
from jax.experimental import pallas as pl
from jax.experimental.pallas import tpu as pltpu
from jax.experimental.pallas import tpu_sc as plsc

_RADIUS = 0.05
_R2 = np.float32(_RADIUS * _RADIUS)
_B, _N, _D = 8, 2048, 3
_WORDS = _N // 32
_NW = 32
_G = 4

_SB = 2
_TB = _B - _SB
_ROW_TILE = 1024

_CHUNKS = _NW // _SB
_ROWS_PER_W = _N // _CHUNKS
_SUPER = _ROWS_PER_W // (16 * _G)
_OUT_PER_W = _ROWS_PER_W * _WORDS


def _bf16_rne(v):
    u = lax.bitcast_convert_type(v, jnp.int32)
    lsb = lax.shift_right_logical(u, 16) & 1
    r = (u + (0x7FFF + lsb)) & jnp.int32(-65536)
    return lax.bitcast_convert_type(r, jnp.float32)


def _sc_body(locs_hbm, out_hbm, locs_v, out_v, p_x2, p_y2, p_z2, p_sq, sem):
    wid = lax.axis_index("s") * 2 + lax.axis_index("c")
    batch = wid // _CHUNKS
    chunk = wid % _CHUNKS
    lbase = (_TB + batch) * (_D * _N)

    pltpu.sync_copy(locs_hbm.at[pl.ds(lbase, _D * _N)], locs_v)

    lane = lax.iota(jnp.int32, 16)

    def prep(i, c0):
        sl = pl.ds(i * 16, 16)
        x = locs_v[pl.ds(i * 16, 16)]
        y = locs_v[pl.ds(_N + i * 16, 16)]
        z = locs_v[pl.ds(2 * _N + i * 16, 16)]
        xb = _bf16_rne(x)
        yb = _bf16_rne(y)
        zb = _bf16_rne(z)
        p_x2[sl] = xb + xb
        p_y2[sl] = yb + yb
        p_z2[sl] = zb + zb
        p_sq[sl] = (x * x + y * y) + z * z
        return c0
    lax.fori_loop(0, _N // 16, prep, 0)

    weights = [np.float32(2.0 ** t) for t in range(32)]

    def super_group(sg, c1):
        xi, yi, zi, sqi, rbase = [], [], [], [], []
        for g in range(_G):
            base = chunk * _ROWS_PER_W + sg * (16 * _G) + g * 16
            xi.append(_bf16_rne(locs_v[pl.ds(base, 16)]))
            yi.append(_bf16_rne(locs_v[pl.ds(_N + base, 16)]))
            zi.append(_bf16_rne(locs_v[pl.ds(2 * _N + base, 16)]))
            sqi.append(p_sq[pl.ds(base, 16)])
            rbase.append((lane + sg * (16 * _G) + g * 16) * _WORDS)

        def word(w, c2):
            j0 = w * 32
            xv = [p_x2[pl.ds(j0, 16)], p_x2[pl.ds(j0 + 16, 16)]]
            yv = [p_y2[pl.ds(j0, 16)], p_y2[pl.ds(j0 + 16, 16)]]
            zv = [p_z2[pl.ds(j0, 16)], p_z2[pl.ds(j0 + 16, 16)]]
            sv = [p_sq[pl.ds(j0, 16)], p_sq[pl.ds(j0 + 16, 16)]]
            acc = [jnp.zeros((16,), jnp.float32) for _ in range(_G)]
            for t in range(32):
                h, l = divmod(t, 16)
                x2 = xv[h][l]
                y2 = yv[h][l]
                z2 = zv[h][l]
                sqj = sv[h][l]
                for g in range(_G):
                    dot2 = (xi[g] * x2 + yi[g] * y2) + zi[g] * z2
                    d2 = (sqi[g] + sqj) - dot2
                    acc[g] = jnp.where(d2 <= _R2, acc[g] + weights[t], acc[g])
            for g in range(_G):
                plsc.store_scatter(out_v, [rbase[g] + w], acc[g])
            return c2
        lax.fori_loop(0, _WORDS, word, 0)
        return c1
    lax.fori_loop(0, _SUPER, super_group, 0)

    pltpu.sync_copy(out_v, out_hbm.at[pl.ds(wid * _OUT_PER_W, _OUT_PER_W)])


def _sc_run(locs_t):
    mesh = plsc.VectorSubcoreMesh(core_axis_name="c", subcore_axis_name="s")
    run = pl.kernel(
        _sc_body,
        out_type=jax.ShapeDtypeStruct((_NW * _OUT_PER_W,), jnp.float32),
        mesh=mesh,
        compiler_params=pltpu.CompilerParams(needs_layout_passes=False),
        scratch_types=[
            pltpu.VMEM((_D * _N,), jnp.float32),
            pltpu.VMEM((_OUT_PER_W,), jnp.float32),
        ] + [pltpu.VMEM((_N,), jnp.float32)] * 4 + [
            pltpu.SemaphoreType.DMA,
        ],
    )
    return run(locs_t).reshape(_SB, _N, _WORDS)


def _tc_body(locs_ref, pack_ref, out_ref):
    rt = pl.program_id(1)
    a = locs_ref[0]
    x = a[0]
    y = a[1]
    z = a[2]
    sq = (x * x + y * y) + z * z
    lhs = locs_ref[0, :, pl.ds(rt * _ROW_TILE, _ROW_TILE)]
    lx = lhs[0]
    ly = lhs[1]
    lz = lhs[2]
    sqi = (lx * lx + ly * ly) + lz * lz
    dots = lax.dot_general(
        lhs.astype(jnp.bfloat16), a.astype(jnp.bfloat16),
        (((0,), (0,)), ((), ())),
        preferred_element_type=jnp.float32)
    d2 = (sqi[:, None] + sq[None, :]) - 2.0 * dots
    mask = (d2 <= _R2).astype(jnp.float32)
    out_ref[0] = lax.dot_general(
        mask, pack_ref[...], (((1,), (0,)), ((), ())),
        preferred_element_type=jnp.float32)


def _tc_run(locs_p, pack):
    grid = (_TB, _N // _ROW_TILE)
    return pl.pallas_call(
        _tc_body,
        grid=grid,
        in_specs=[
            pl.BlockSpec((1, _D, _N), lambda b, rt: (b, 0, 0)),
            pl.BlockSpec((_N, _WORDS), lambda b, rt: (0, 0)),
        ],
        out_specs=pl.BlockSpec((1, _ROW_TILE, _WORDS),
                               lambda b, rt: (b, rt, 0)),
        out_shape=jax.ShapeDtypeStruct((_TB, _N, _WORDS), jnp.float32),
    )(locs_p, pack)


def _pack_matrix():
    j = np.arange(_N)
    p = np.zeros((_N, _WORDS), np.float32)
    p[j, j // 32] = (2.0 ** (j % 32)).astype(np.float32)
    return p


_PACK = _pack_matrix()


def kernel(locs):
    locs_p = jnp.transpose(locs, (0, 2, 1))
    out_sc = _sc_run(locs_p.reshape(_B * _D * _N))
    out_tc = _tc_run(locs_p, jnp.asarray(_PACK))
    return jnp.concatenate([out_tc, out_sc], axis=0)


# speedup vs baseline: 1.5800x; 1.1196x over previous
import functools

import jax
import jax.numpy as jnp
import numpy as np
from jax import lax
from jax.experimental import pallas as pl
from jax.experimental.pallas import tpu as pltpu
from jax.experimental.pallas import tpu_sc as plsc

_RADIUS = 0.05
_R2 = np.float32(_RADIUS * _RADIUS)
_B, _N, _D = 8, 2048, 3
_WORDS = _N // 32
_NW = 32
_G = 4

_SB = 1
_TB = _B - _SB
_ROW_TILE = 1024

_CHUNKS = _NW // _SB
_ROWS_PER_W = _N // _CHUNKS
_SUPER = _ROWS_PER_W // (16 * _G)
_OUT_PER_W = _ROWS_PER_W * _WORDS


def _bf16_rne(v):
    u = lax.bitcast_convert_type(v, jnp.int32)
    lsb = lax.shift_right_logical(u, 16) & 1
    r = (u + (0x7FFF + lsb)) & jnp.int32(-65536)
    return lax.bitcast_convert_type(r, jnp.float32)


def _sc_body(locs_hbm, out_hbm, locs_v, out_v, p_x2, p_y2, p_z2, p_sq, sem):
    wid = lax.axis_index("s") * 2 + lax.axis_index("c")
    batch = wid // _CHUNKS
    chunk = wid % _CHUNKS
    lbase = (_TB + batch) * (_D * _N)

    pltpu.sync_copy(locs_hbm.at[pl.ds(lbase, _D * _N)], locs_v)

    lane = lax.iota(jnp.int32, 16)

    def prep(i, c0):
        sl = pl.ds(i * 16, 16)
        x = locs_v[pl.ds(i * 16, 16)]
        y = locs_v[pl.ds(_N + i * 16, 16)]
        z = locs_v[pl.ds(2 * _N + i * 16, 16)]
        xb = _bf16_rne(x)
        yb = _bf16_rne(y)
        zb = _bf16_rne(z)
        p_x2[sl] = xb + xb
        p_y2[sl] = yb + yb
        p_z2[sl] = zb + zb
        p_sq[sl] = (x * x + y * y) + z * z
        return c0
    lax.fori_loop(0, _N // 16, prep, 0)

    weights = [np.float32(2.0 ** t) for t in range(32)]

    def super_group(sg, c1):
        xi, yi, zi, sqi, rbase = [], [], [], [], []
        for g in range(_G):
            base = chunk * _ROWS_PER_W + sg * (16 * _G) + g * 16
            xi.append(_bf16_rne(locs_v[pl.ds(base, 16)]))
            yi.append(_bf16_rne(locs_v[pl.ds(_N + base, 16)]))
            zi.append(_bf16_rne(locs_v[pl.ds(2 * _N + base, 16)]))
            sqi.append(p_sq[pl.ds(base, 16)])
            rbase.append((lane + sg * (16 * _G) + g * 16) * _WORDS)

        def word(w, c2):
            j0 = w * 32
            xv = [p_x2[pl.ds(j0, 16)], p_x2[pl.ds(j0 + 16, 16)]]
            yv = [p_y2[pl.ds(j0, 16)], p_y2[pl.ds(j0 + 16, 16)]]
            zv = [p_z2[pl.ds(j0, 16)], p_z2[pl.ds(j0 + 16, 16)]]
            sv = [p_sq[pl.ds(j0, 16)], p_sq[pl.ds(j0 + 16, 16)]]
            acc = [jnp.zeros((16,), jnp.float32) for _ in range(_G)]
            for t in range(32):
                h, l = divmod(t, 16)
                x2 = xv[h][l]
                y2 = yv[h][l]
                z2 = zv[h][l]
                sqj = sv[h][l]
                for g in range(_G):
                    dot2 = (xi[g] * x2 + yi[g] * y2) + zi[g] * z2
                    d2 = (sqi[g] + sqj) - dot2
                    acc[g] = jnp.where(d2 <= _R2, acc[g] + weights[t], acc[g])
            for g in range(_G):
                plsc.store_scatter(out_v, [rbase[g] + w], acc[g])
            return c2
        lax.fori_loop(0, _WORDS, word, 0)
        return c1
    lax.fori_loop(0, _SUPER, super_group, 0)

    pltpu.sync_copy(out_v, out_hbm.at[pl.ds(wid * _OUT_PER_W, _OUT_PER_W)])


def _sc_run(locs_t):
    mesh = plsc.VectorSubcoreMesh(core_axis_name="c", subcore_axis_name="s")
    run = pl.kernel(
        _sc_body,
        out_type=jax.ShapeDtypeStruct((_NW * _OUT_PER_W,), jnp.float32),
        mesh=mesh,
        compiler_params=pltpu.CompilerParams(needs_layout_passes=False),
        scratch_types=[
            pltpu.VMEM((_D * _N,), jnp.float32),
            pltpu.VMEM((_OUT_PER_W,), jnp.float32),
        ] + [pltpu.VMEM((_N,), jnp.float32)] * 4 + [
            pltpu.SemaphoreType.DMA,
        ],
    )
    return run(locs_t).reshape(_SB, _N, _WORDS)


def _tc_body(locs_ref, pack_ref, out_ref):
    rt = pl.program_id(1)
    a = locs_ref[0]
    x = a[0]
    y = a[1]
    z = a[2]
    sq = (x * x + y * y) + z * z
    lhs = locs_ref[0, :, pl.ds(rt * _ROW_TILE, _ROW_TILE)]
    lx = lhs[0]
    ly = lhs[1]
    lz = lhs[2]
    sqi = (lx * lx + ly * ly) + lz * lz
    dots = lax.dot_general(
        lhs.astype(jnp.bfloat16), a.astype(jnp.bfloat16),
        (((0,), (0,)), ((), ())),
        preferred_element_type=jnp.float32)
    d2 = (sqi[:, None] + sq[None, :]) - 2.0 * dots
    mask = (d2 <= _R2).astype(jnp.float32)
    out_ref[0] = lax.dot_general(
        mask, pack_ref[...], (((1,), (0,)), ((), ())),
        preferred_element_type=jnp.float32)


def _tc_run(locs_p, pack):
    grid = (_TB, _N // _ROW_TILE)
    return pl.pallas_call(
        _tc_body,
        grid=grid,
        in_specs=[
            pl.BlockSpec((1, _D, _N), lambda b, rt: (b, 0, 0)),
            pl.BlockSpec((_N, _WORDS), lambda b, rt: (0, 0)),
        ],
        out_specs=pl.BlockSpec((1, _ROW_TILE, _WORDS),
                               lambda b, rt: (b, rt, 0)),
        out_shape=jax.ShapeDtypeStruct((_TB, _N, _WORDS), jnp.float32),
    )(locs_p, pack)


def _pack_matrix():
    j = np.arange(_N)
    p = np.zeros((_N, _WORDS), np.float32)
    p[j, j // 32] = (2.0 ** (j % 32)).astype(np.float32)
    return p


_PACK = _pack_matrix()


def kernel(locs):
    locs_p = jnp.transpose(locs, (0, 2, 1))
    out_sc = _sc_run(locs_p.reshape(_B * _D * _N))
    out_tc = _tc_run(locs_p, jnp.asarray(_PACK))
    return jnp.concatenate([out_tc, out_sc], axis=0)


# speedup vs baseline: 1.7582x; 1.1128x over previous
import functools

import jax
import jax.numpy as jnp
import numpy as np
from jax import lax
from jax.experimental import pallas as pl
from jax.experimental.pallas import tpu as pltpu
from jax.experimental.pallas import tpu_sc as plsc

_RADIUS = 0.05
_R2 = np.float32(_RADIUS * _RADIUS)
_B, _N, _D = 8, 2048, 3
_WORDS = _N // 32
_NW = 32
_G = 4

_SB = 1
_TB = _B - _SB
_ROW_TILE = 2048

_CHUNKS = _NW // _SB
_ROWS_PER_W = _N // _CHUNKS
_SUPER = _ROWS_PER_W // (16 * _G)
_OUT_PER_W = _ROWS_PER_W * _WORDS


def _bf16_rne(v):
    u = lax.bitcast_convert_type(v, jnp.int32)
    lsb = lax.shift_right_logical(u, 16) & 1
    r = (u + (0x7FFF + lsb)) & jnp.int32(-65536)
    return lax.bitcast_convert_type(r, jnp.float32)


def _sc_body(locs_hbm, out_hbm, locs_v, out_v, p_x2, p_y2, p_z2, p_sq, sem):
    wid = lax.axis_index("s") * 2 + lax.axis_index("c")
    batch = wid // _CHUNKS
    chunk = wid % _CHUNKS
    lbase = (_TB + batch) * (_D * _N)

    pltpu.sync_copy(locs_hbm.at[pl.ds(lbase, _D * _N)], locs_v)

    lane = lax.iota(jnp.int32, 16)

    def prep(i, c0):
        sl = pl.ds(i * 16, 16)
        x = locs_v[pl.ds(i * 16, 16)]
        y = locs_v[pl.ds(_N + i * 16, 16)]
        z = locs_v[pl.ds(2 * _N + i * 16, 16)]
        xb = _bf16_rne(x)
        yb = _bf16_rne(y)
        zb = _bf16_rne(z)
        p_x2[sl] = xb + xb
        p_y2[sl] = yb + yb
        p_z2[sl] = zb + zb
        p_sq[sl] = (x * x + y * y) + z * z
        return c0
    lax.fori_loop(0, _N // 16, prep, 0)

    weights = [np.float32(2.0 ** t) for t in range(32)]

    def super_group(sg, c1):
        xi, yi, zi, sqi, rbase = [], [], [], [], []
        for g in range(_G):
            base = chunk * _ROWS_PER_W + sg * (16 * _G) + g * 16
            xi.append(_bf16_rne(locs_v[pl.ds(base, 16)]))
            yi.append(_bf16_rne(locs_v[pl.ds(_N + base, 16)]))
            zi.append(_bf16_rne(locs_v[pl.ds(2 * _N + base, 16)]))
            sqi.append(p_sq[pl.ds(base, 16)])
            rbase.append((lane + sg * (16 * _G) + g * 16) * _WORDS)

        def word(w, c2):
            j0 = w * 32
            xv = [p_x2[pl.ds(j0, 16)], p_x2[pl.ds(j0 + 16, 16)]]
            yv = [p_y2[pl.ds(j0, 16)], p_y2[pl.ds(j0 + 16, 16)]]
            zv = [p_z2[pl.ds(j0, 16)], p_z2[pl.ds(j0 + 16, 16)]]
            sv = [p_sq[pl.ds(j0, 16)], p_sq[pl.ds(j0 + 16, 16)]]
            acc = [jnp.zeros((16,), jnp.float32) for _ in range(_G)]
            for t in range(32):
                h, l = divmod(t, 16)
                x2 = xv[h][l]
                y2 = yv[h][l]
                z2 = zv[h][l]
                sqj = sv[h][l]
                for g in range(_G):
                    dot2 = (xi[g] * x2 + yi[g] * y2) + zi[g] * z2
                    d2 = (sqi[g] + sqj) - dot2
                    acc[g] = jnp.where(d2 <= _R2, acc[g] + weights[t], acc[g])
            for g in range(_G):
                plsc.store_scatter(out_v, [rbase[g] + w], acc[g])
            return c2
        lax.fori_loop(0, _WORDS, word, 0)
        return c1
    lax.fori_loop(0, _SUPER, super_group, 0)

    pltpu.sync_copy(out_v, out_hbm.at[pl.ds(wid * _OUT_PER_W, _OUT_PER_W)])


def _sc_run(locs_t):
    mesh = plsc.VectorSubcoreMesh(core_axis_name="c", subcore_axis_name="s")
    run = pl.kernel(
        _sc_body,
        out_type=jax.ShapeDtypeStruct((_NW * _OUT_PER_W,), jnp.float32),
        mesh=mesh,
        compiler_params=pltpu.CompilerParams(needs_layout_passes=False),
        scratch_types=[
            pltpu.VMEM((_D * _N,), jnp.float32),
            pltpu.VMEM((_OUT_PER_W,), jnp.float32),
        ] + [pltpu.VMEM((_N,), jnp.float32)] * 4 + [
            pltpu.SemaphoreType.DMA,
        ],
    )
    return run(locs_t).reshape(_SB, _N, _WORDS)


def _tc_body(locs_ref, pack_ref, out_ref):
    rt = pl.program_id(1)
    a = locs_ref[0]
    x = a[0]
    y = a[1]
    z = a[2]
    sq = (x * x + y * y) + z * z
    lhs = locs_ref[0, :, pl.ds(rt * _ROW_TILE, _ROW_TILE)]
    lx = lhs[0]
    ly = lhs[1]
    lz = lhs[2]
    sqi = (lx * lx + ly * ly) + lz * lz
    dots = lax.dot_general(
        lhs.astype(jnp.bfloat16), a.astype(jnp.bfloat16),
        (((0,), (0,)), ((), ())),
        preferred_element_type=jnp.float32)
    d2 = (sqi[:, None] + sq[None, :]) - 2.0 * dots
    mask = (d2 <= _R2).astype(jnp.float32)
    out_ref[0] = lax.dot_general(
        mask, pack_ref[...], (((1,), (0,)), ((), ())),
        preferred_element_type=jnp.float32)


def _tc_run(locs_p, pack):
    grid = (_TB, _N // _ROW_TILE)
    return pl.pallas_call(
        _tc_body,
        grid=grid,
        in_specs=[
            pl.BlockSpec((1, _D, _N), lambda b, rt: (b, 0, 0)),
            pl.BlockSpec((_N, _WORDS), lambda b, rt: (0, 0)),
        ],
        out_specs=pl.BlockSpec((1, _ROW_TILE, _WORDS),
                               lambda b, rt: (b, rt, 0)),
        out_shape=jax.ShapeDtypeStruct((_B, _N, _WORDS), jnp.float32),
    )(locs_p, pack)


def _pack_matrix():
    j = np.arange(_N)
    p = np.zeros((_N, _WORDS), np.float32)
    p[j, j // 32] = (2.0 ** (j % 32)).astype(np.float32)
    return p


_PACK = _pack_matrix()


def kernel(locs):
    locs_p = jnp.transpose(locs, (0, 2, 1))
    out_sc = _sc_run(locs_p.reshape(_B * _D * _N))
    out_tc = _tc_run(locs_p, jnp.asarray(_PACK))
    return lax.dynamic_update_slice(out_tc, out_sc, (_TB, 0, 0))
